# Initial kernel scaffold; baseline (speedup 1.0000x reference)
import jax
import jax.numpy as jnp
from jax.experimental import pallas as pl


def kernel(x, edge_index, edge_weight, W):
    raise NotImplementedError("write your pallas kernel here")


# speedup vs baseline: 8.2427x; 8.2427x over previous
import functools

import jax
import jax.numpy as jnp
from jax import lax
from jax.experimental import pallas as pl
from jax.experimental.pallas import tpu as pltpu
from jax.experimental.pallas import tpu_sc as plsc

N_NODES = 10000
N_EDGES = 320000
D = 128

NC = 2
NS = 16
L = 16
NW = NC * NS

EPW = N_EDGES // NW
K = 80
NCHUNK = EPW // K
RPT = 624
REM = N_NODES - NS * RPT
ZR = 48

_mesh = plsc.VectorSubcoreMesh(core_axis_name="c", subcore_axis_name="s")


@functools.partial(
    pl.kernel,
    out_type=jax.ShapeDtypeStruct((NC, N_NODES, D), jnp.float32),
    mesh=_mesh,
    scratch_types=[
        pltpu.VMEM((2, K), jnp.int32),
        pltpu.VMEM((2, K), jnp.int32),
        pltpu.VMEM((2, K), jnp.float32),
        pltpu.VMEM((2, K, D), jnp.float32),
        pltpu.VMEM((ZR, D), jnp.float32),
        pltpu.VMEM_SHARED((N_NODES, D), jnp.float32),
        pltpu.SemaphoreType.DMA,
        pltpu.SemaphoreType.DMA,
        pltpu.SemaphoreType.DMA,
        pltpu.SemaphoreType.DMA,
        pltpu.SemaphoreType.DMA,
        pltpu.SemaphoreType.DMA,
    ],
)
def _sc_aggregate(x_hbm, src_hbm, dst_hbm, w_hbm, out_hbm,
                  src_v, dst_v, w_v, rows_v, zero_v, acc_sh,
                  sem_ld0, sem_ld1, sem_g0, sem_g1, sem_s0, sem_s1):
    c = lax.axis_index("c")
    s = lax.axis_index("s")
    wid = c * NS + s
    sem_ld = (sem_ld0, sem_ld1)
    sem_g = (sem_g0, sem_g1)
    sem_s = (sem_s0, sem_s1)

    zeros = jnp.zeros((L,), jnp.float32)

    def zero_row(i, _):
        for j in range(D // L):
            zero_v[i, pl.ds(j * L, L)] = zeros
        return 0

    lax.fori_loop(0, ZR, zero_row, 0)
    for b in range(RPT // ZR):
        pltpu.sync_copy(zero_v, acc_sh.at[pl.ds(s * RPT + b * ZR, ZR)])

    @pl.when(s == NS - 1)
    def _zero_tail():
        pltpu.sync_copy(zero_v.at[pl.ds(0, REM)],
                        acc_sh.at[pl.ds(NS * RPT, REM)])

    plsc.subcore_barrier()

    def issue_load(ci, b):
        pltpu.async_copy(src_hbm.at[wid, ci], src_v.at[b], sem_ld[b])
        pltpu.async_copy(dst_hbm.at[wid, ci], dst_v.at[b], sem_ld[b])
        pltpu.async_copy(w_hbm.at[wid, ci], w_v.at[b], sem_ld[b])

    def wait_load(b):
        pltpu.make_async_copy(src_hbm.at[wid, 0], src_v.at[b],
                              sem_ld[b]).wait()
        pltpu.make_async_copy(dst_hbm.at[wid, 0], dst_v.at[b],
                              sem_ld[b]).wait()
        pltpu.make_async_copy(w_hbm.at[wid, 0], w_v.at[b], sem_ld[b]).wait()

    def wait_scatter(b):
        pltpu.make_async_copy(rows_v.at[b], acc_sh.at[dst_v.at[b]],
                              sem_s[b]).wait()

    def scale(b):
        def group_body(g, _):
            wvec = w_v[b, pl.ds(g * L, L)]
            for kk in range(L):
                wk = wvec[kk]
                k = g * L + kk
                for j in range(D // L):
                    sl = pl.ds(j * L, L)
                    rows_v[b, k, sl] = rows_v[b, k, sl] * wk
            return 0

        lax.fori_loop(0, K // L, group_body, 0)

    def chunk(ci, b, first):
        wait_load(b)
        gather = pltpu.async_copy(x_hbm.at[src_v.at[b]], rows_v.at[b],
                                  sem_g[b])
        if not first:
            wait_scatter(1 - b)
        issue_load(ci + 1, 1 - b)
        gather.wait()
        scale(b)
        pltpu.async_copy(rows_v.at[b], acc_sh.at[dst_v.at[b]], sem_s[b],
                         add=True)

    issue_load(jnp.int32(0), 0)
    chunk(jnp.int32(0), 0, first=True)
    chunk(jnp.int32(1), 1, first=False)

    def loop_body(t, _):
        for b in range(2):
            chunk(2 * t + b, b, first=False)
        return 0

    lax.fori_loop(1, NCHUNK // 2, loop_body, 0)

    wait_load(0)
    pltpu.async_copy(x_hbm.at[src_v.at[0]], rows_v.at[0], sem_g0)
    wait_scatter(1)
    pltpu.make_async_copy(x_hbm.at[src_v.at[0]], rows_v.at[0], sem_g0).wait()
    scale(0)
    pltpu.async_copy(rows_v.at[0], acc_sh.at[dst_v.at[0]], sem_s0, add=True)
    wait_scatter(0)
    plsc.subcore_barrier()

    pltpu.sync_copy(acc_sh.at[pl.ds(s * RPT, RPT)],
                    out_hbm.at[c, pl.ds(s * RPT, RPT)])

    @pl.when(s == NS - 1)
    def _write_tail():
        pltpu.sync_copy(acc_sh.at[pl.ds(NS * RPT, REM)],
                        out_hbm.at[c, pl.ds(NS * RPT, REM)])


def _tc_body(p_ref, w_ref, o_ref):
    agg = p_ref[0] + p_ref[1]
    o_ref[...] = lax.dot_general(agg, w_ref[...], (((1,), (1,)), ((), ())),
                                 preferred_element_type=jnp.float32)


_tc_linear = pl.pallas_call(
    _tc_body,
    out_shape=jax.ShapeDtypeStruct((N_NODES, D), jnp.float32),
)


def kernel(x, edge_index, edge_weight, W):
    src = edge_index[0].astype(jnp.int32).reshape(NW, NCHUNK, K)
    dst = edge_index[1].astype(jnp.int32).reshape(NW, NCHUNK, K)
    w = edge_weight.astype(jnp.float32).reshape(NW, NCHUNK, K)
    partials = _sc_aggregate(x, src, dst, w)
    return _tc_linear(partials, W)
